# Initial kernel scaffold; baseline (speedup 1.0000x reference)
import jax
import jax.numpy as jnp
from jax.experimental import pallas as pl


def kernel(x, edge_index, edge_attr, W1, b1, W2, b2, tag0_W, tag0_b, tag1_W, tag1_b, tag2_W, tag2_b):
    raise NotImplementedError("write your pallas kernel here")


# speedup vs baseline: 5.0351x; 5.0351x over previous
import functools
import jax
import jax.numpy as jnp
from jax import lax
from jax.experimental import pallas as pl
from jax.experimental.pallas import tpu as pltpu
from jax.experimental.pallas import tpu_sc as plsc

N = 10000
E = 320000
DN = 128
DE = 16
DH = 64
DW = 128
DA = 80
NC = 2
NS = 16
L = 16
NW = NC * NS
PAD = 7680
EH = E + PAD
E2P = 2 * EH
PER_W = E2P // NW
IR_W = PER_W // 128
N_ACC = 10240
ROWS_PER_TILE = N_ACC // NS
CHUNK = 1024
NCHUNK = PER_W // CHUNK

_f32 = jnp.float32


def _tc_isdir(ei3):
    def body(ei_ref, o_ref):
        src0 = ei_ref[0, 0, 0]
        tgt0 = ei_ref[1, 0, 0]
        m = jnp.logical_and(ei_ref[0] == tgt0, ei_ref[1] == src0)
        red = jnp.max(jnp.where(m, 1.0, 0.0))
        o_ref[...] = (1.0 - red).reshape(1, 1)

    return pl.pallas_call(
        body, out_shape=jax.ShapeDtypeStruct((1, 1), _f32))(ei3)


def _tc_nodes(x, w1a, w1b, isd):
    def body(x_ref, wa_ref, wb_ref, isd_ref, xi_out, xj_out):
        s = isd_ref[0, 0]
        xi = jnp.dot(x_ref[...], wa_ref[...], preferred_element_type=_f32)
        xj = jnp.dot(x_ref[...], wb_ref[...], preferred_element_type=_f32)
        z = jnp.zeros((N, DW - DH), _f32)
        xi_out[pl.ds(0, N), pl.ds(0, DH)] = xi
        xi_out[pl.ds(N, N), pl.ds(0, DH)] = xi * s
        xi_out[pl.ds(0, N), pl.ds(DH, DW - DH)] = z
        xi_out[pl.ds(N, N), pl.ds(DH, DW - DH)] = z
        xj_out[pl.ds(0, N), pl.ds(0, DH)] = xj
        xj_out[pl.ds(N, N), pl.ds(0, DH)] = xj * s
        xj_out[pl.ds(0, N), pl.ds(DH, DW - DH)] = z
        xj_out[pl.ds(N, N), pl.ds(DH, DW - DH)] = z

    return pl.pallas_call(
        body,
        in_specs=[
            pl.BlockSpec(memory_space=pltpu.VMEM),
            pl.BlockSpec(memory_space=pltpu.VMEM),
            pl.BlockSpec(memory_space=pltpu.VMEM),
            pl.BlockSpec(memory_space=pltpu.SMEM),
        ],
        out_specs=[pl.BlockSpec(memory_space=pltpu.VMEM),
                   pl.BlockSpec(memory_space=pltpu.VMEM)],
        out_shape=[jax.ShapeDtypeStruct((2 * N, DW), _f32),
                   jax.ShapeDtypeStruct((2 * N, DW), _f32)],
    )(x, w1a, w1b, isd)


def _tc_ae(isd, ea_pad, wef, b1r):
    blk = 4096
    nblk = EH // blk

    def body(isd_ref, ea_ref, w_ref, b_ref, o_ref):
        h = pl.program_id(0)
        s = jnp.where(h == 0, 1.0, isd_ref[0, 0])
        lo = (jnp.dot(ea_ref[...], w_ref[0],
                      preferred_element_type=_f32) + b_ref[...]) * s
        wcol = jnp.where(
            lax.broadcasted_iota(jnp.int32, (blk, DA - DH), 1) == 0, s, 0.0)
        o_ref[0, :, pl.ds(0, DH)] = lo
        o_ref[0, :, pl.ds(DH, DA - DH)] = wcol

    return pl.pallas_call(
        body,
        grid=(2, nblk),
        in_specs=[
            pl.BlockSpec(memory_space=pltpu.SMEM),
            pl.BlockSpec((blk, DE), lambda h, j: (j, 0)),
            pl.BlockSpec((1, DE, DH), lambda h, j: (h, 0, 0)),
            pl.BlockSpec((1, DH), lambda h, j: (0, 0)),
        ],
        out_specs=pl.BlockSpec((1, blk, DW), lambda h, j: (h, j, 0)),
        out_shape=jax.ShapeDtypeStruct((2, EH, DW), _f32),
    )(isd, ea_pad, wef, b1r)


def _tc_postagg(s_part, w2, b2r, w0, isd):
    def body(s_ref, w2_ref, b2_ref, w0_ref, isd_ref,
             acc_out, us_out, dis_out):
        sv = s_ref[...]
        S = sv[0, :N, :DH] + sv[1, :N, :DH]
        deg = sv[0, :N, DH:DH + 1] + sv[1, :N, DH:DH + 1]
        h = jnp.dot(S, w2_ref[...], preferred_element_type=_f32) \
            + deg * b2_ref[...]
        dis = jnp.where(deg > 0, lax.rsqrt(jnp.where(deg > 0, deg, 1.0)), 0.0)
        acc_out[...] = jnp.dot(h, w0_ref[...], preferred_element_type=_f32)
        u = dis * h
        z = jnp.zeros((N, DW - DH), _f32)
        us_out[pl.ds(0, N), pl.ds(0, DH)] = u
        us_out[pl.ds(N, N), pl.ds(0, DH)] = u * isd_ref[0, 0]
        us_out[pl.ds(0, N), pl.ds(DH, DW - DH)] = z
        us_out[pl.ds(N, N), pl.ds(DH, DW - DH)] = z
        dis_out[...] = dis

    return pl.pallas_call(
        body,
        in_specs=[pl.BlockSpec(memory_space=pltpu.VMEM)] * 4
        + [pl.BlockSpec(memory_space=pltpu.SMEM)],
        out_specs=[pl.BlockSpec(memory_space=pltpu.VMEM)] * 3,
        out_shape=[jax.ShapeDtypeStruct((N, DH), _f32),
                   jax.ShapeDtypeStruct((2 * N, DW), _f32),
                   jax.ShapeDtypeStruct((N, 1), _f32)],
    )(s_part, w2, b2r, w0, isd)


def _tc_tag_mid(s_part, dis, wk, acc, isd):
    def body(s_ref, dis_ref, wk_ref, acc_ref, isd_ref, acc_out, us_out):
        sv = s_ref[...]
        dv = dis_ref[...]
        hk = dv * (sv[0, :N, :DH] + sv[1, :N, :DH])
        acc_out[...] = acc_ref[...] + jnp.dot(
            hk, wk_ref[...], preferred_element_type=_f32)
        u = dv * hk
        z = jnp.zeros((N, DW - DH), _f32)
        us_out[pl.ds(0, N), pl.ds(0, DH)] = u
        us_out[pl.ds(N, N), pl.ds(0, DH)] = u * isd_ref[0, 0]
        us_out[pl.ds(0, N), pl.ds(DH, DW - DH)] = z
        us_out[pl.ds(N, N), pl.ds(DH, DW - DH)] = z

    return pl.pallas_call(
        body,
        in_specs=[pl.BlockSpec(memory_space=pltpu.VMEM)] * 4
        + [pl.BlockSpec(memory_space=pltpu.SMEM)],
        out_specs=[pl.BlockSpec(memory_space=pltpu.VMEM)] * 2,
        out_shape=[jax.ShapeDtypeStruct((N, DH), _f32),
                   jax.ShapeDtypeStruct((2 * N, DW), _f32)],
    )(s_part, dis, wk, acc, isd)


def _tc_tag_last(s_part, dis, wk, acc, br, w0n, isd):
    def body(s_ref, dis_ref, wk_ref, acc_ref, b_ref, w0_ref, isd_ref,
             acc_out, us_out):
        sv = s_ref[...]
        dv = dis_ref[...]
        hk = dv * (sv[0, :N, :DH] + sv[1, :N, :DH])
        g = acc_ref[...] + jnp.dot(hk, wk_ref[...],
                                   preferred_element_type=_f32) + b_ref[...]
        hn = jnp.maximum(g, 0.0)
        acc_out[...] = jnp.dot(hn, w0_ref[...], preferred_element_type=_f32)
        u = dv * hn
        z = jnp.zeros((N, DW - DH), _f32)
        us_out[pl.ds(0, N), pl.ds(0, DH)] = u
        us_out[pl.ds(N, N), pl.ds(0, DH)] = u * isd_ref[0, 0]
        us_out[pl.ds(0, N), pl.ds(DH, DW - DH)] = z
        us_out[pl.ds(N, N), pl.ds(DH, DW - DH)] = z

    return pl.pallas_call(
        body,
        in_specs=[pl.BlockSpec(memory_space=pltpu.VMEM)] * 6
        + [pl.BlockSpec(memory_space=pltpu.SMEM)],
        out_specs=[pl.BlockSpec(memory_space=pltpu.VMEM)] * 2,
        out_shape=[jax.ShapeDtypeStruct((N, DH), _f32),
                   jax.ShapeDtypeStruct((2 * N, DW), _f32)],
    )(s_part, dis, wk, acc, br, w0n, isd)


def _tc_tag_final(s_part, dis, wk, acc, br):
    def body(s_ref, dis_ref, wk_ref, acc_ref, b_ref, o_ref):
        sv = s_ref[...]
        hk = dis_ref[...] * (sv[0, :N, :DH] + sv[1, :N, :DH])
        o_ref[...] = acc_ref[...] + jnp.dot(
            hk, wk_ref[...], preferred_element_type=_f32) + b_ref[...]

    return pl.pallas_call(
        body,
        in_specs=[pl.BlockSpec(memory_space=pltpu.VMEM)] * 5,
        out_specs=pl.BlockSpec(memory_space=pltpu.VMEM),
        out_shape=jax.ShapeDtypeStruct((N, DH), _f32),
    )(s_part, dis, wk, acc, br)


def _mesh():
    return plsc.VectorSubcoreMesh(
        core_axis_name="c", subcore_axis_name="s")


def _zero_acc(buf, acc, tz):
    def zrow(r, _):
        for sl in range(DW // L):
            buf[r, pl.ds(sl * L, L)] = jnp.zeros((L,), _f32)
        return 0
    lax.fori_loop(0, 128, zrow, 0, unroll=8)
    for j in range(ROWS_PER_TILE // 128):
        pltpu.sync_copy(buf, acc.at[pl.ds(tz + j * 128, 128)])


def _sc_agg(xi, xj, ae, ga, gb, sc2):
    @functools.partial(
        pl.kernel,
        out_type=jax.ShapeDtypeStruct((NC, N_ACC, DW), _f32),
        mesh=_mesh(),
        scratch_types=[
            pltpu.VMEM_SHARED((N_ACC, DW), _f32),
            pltpu.VMEM((8, 128), jnp.int32),
            pltpu.VMEM((8, 128), jnp.int32),
            pltpu.VMEM((8, 128), jnp.int32),
            pltpu.VMEM((128, DW), _f32),
            pltpu.VMEM((128, DW), _f32),
            pltpu.VMEM((64, DW), _f32),
            pltpu.SemaphoreType.DMA,
            pltpu.SemaphoreType.DMA,
        ],
    )
    def k(xi_h, xj_h, ae_h, ga_h, gb_h, sc2_h, s_out,
          s_acc, idxa, idxb, scx, rowsa, rowsb, aebuf, sem1, sem2):
        cid = lax.axis_index("c")
        sid = lax.axis_index("s")
        wid = cid * NS + sid
        base_row = wid * IR_W
        tz = sid * ROWS_PER_TILE

        _zero_acc(rowsa, s_acc, tz)
        plsc.subcore_barrier()

        def chunk(i, _):
            crow = base_row + i * (CHUNK // 128)
            pltpu.sync_copy(ga_h.at[pl.ds(crow, CHUNK // 128)], idxa)
            pltpu.sync_copy(gb_h.at[pl.ds(crow, CHUNK // 128)], idxb)
            pltpu.sync_copy(sc2_h.at[pl.ds(crow, CHUNK // 128)], scx)
            eoff = wid * PER_W + i * CHUNK
            for j in range(CHUNK // 128):
                d1 = pltpu.async_copy(xi_h.at[idxa.at[j]], rowsa, sem1)
                d2 = pltpu.async_copy(xj_h.at[idxb.at[j]], rowsb, sem2)
                d1.wait()
                d2.wait()

                for half in range(2):
                    pltpu.sync_copy(
                        ae_h.at[pl.ds(eoff + j * 128 + half * 64, 64)], aebuf)

                    def comp(r, _):
                        rr = half * 64 + r
                        for sl in range(DW // L):
                            s_ = pl.ds(sl * L, L)
                            rowsa[rr, s_] = jnp.maximum(
                                rowsa[rr, s_] + rowsb[rr, s_] + aebuf[r, s_],
                                0.0)
                        return 0
                    lax.fori_loop(0, 64, comp, 0, unroll=4)

                pltpu.sync_copy(rowsa, s_acc.at[scx.at[j]], add=True)
            return 0
        lax.fori_loop(0, NCHUNK, chunk, 0)

        plsc.subcore_barrier()
        pltpu.sync_copy(s_acc.at[pl.ds(tz, ROWS_PER_TILE)],
                        s_out.at[cid, pl.ds(tz, ROWS_PER_TILE)])

    return k(xi, xj, ae, ga, gb, sc2)


def _sc_spmv(us, gb, sc2):
    @functools.partial(
        pl.kernel,
        out_type=jax.ShapeDtypeStruct((NC, N_ACC, DW), _f32),
        mesh=_mesh(),
        scratch_types=[
            pltpu.VMEM_SHARED((N_ACC, DW), _f32),
            pltpu.VMEM((8, 128), jnp.int32),
            pltpu.VMEM((8, 128), jnp.int32),
            pltpu.VMEM((128, DW), _f32),
            pltpu.VMEM((128, DW), _f32),
            pltpu.SemaphoreType.DMA,
            pltpu.SemaphoreType.DMA,
        ],
    )
    def k(us_h, gb_h, sc2_h, s_out,
          s_acc, idxb, scx, rows0, rows1, sem0, sem1):
        cid = lax.axis_index("c")
        sid = lax.axis_index("s")
        wid = cid * NS + sid
        base_row = wid * IR_W
        tz = sid * ROWS_PER_TILE

        _zero_acc(rows0, s_acc, tz)
        plsc.subcore_barrier()

        def chunk(i, _):
            crow = base_row + i * (CHUNK // 128)
            pltpu.sync_copy(gb_h.at[pl.ds(crow, CHUNK // 128)], idxb)
            pltpu.sync_copy(sc2_h.at[pl.ds(crow, CHUNK // 128)], scx)
            bufs = (rows0, rows1)
            sems = (sem0, sem1)
            d0 = pltpu.async_copy(us_h.at[idxb.at[0]], rows0, sem0)
            dmas = [d0, None]
            for j in range(CHUNK // 128):
                b = j % 2
                nb = (j + 1) % 2
                if j + 1 < CHUNK // 128:
                    dmas[nb] = pltpu.async_copy(
                        us_h.at[idxb.at[j + 1]], bufs[nb], sems[nb])
                dmas[b].wait()
                pltpu.sync_copy(bufs[b], s_acc.at[scx.at[j]], add=True)
            return 0
        lax.fori_loop(0, NCHUNK, chunk, 0)

        plsc.subcore_barrier()
        pltpu.sync_copy(s_acc.at[pl.ds(tz, ROWS_PER_TILE)],
                        s_out.at[cid, pl.ds(tz, ROWS_PER_TILE)])

    return k(us, gb, sc2)


def kernel(x, edge_index, edge_attr, W1, b1, W2, b2,
           tag0_W, tag0_b, tag1_W, tag1_b, tag2_W, tag2_b):
    row, col = edge_index[0], edge_index[1]
    ei3 = edge_index.reshape(2, E // 128, 128)
    isd = _tc_isdir(ei3)

    w1a, w1b, w1e = W1[:DN], W1[DN:2 * DN], W1[2 * DN:]
    sgn = jnp.array([-1.0, 1.0, -1.0] + [1.0] * (DE - 3), _f32)[:, None]
    wef = jnp.stack([w1e, w1e * sgn])
    xi_s, xj_s = _tc_nodes(x, w1a, w1b, isd)
    ea_pad = jnp.zeros((EH, DE), _f32).at[:E].set(edge_attr)
    ae = _tc_ae(isd, ea_pad, wef, b1.reshape(1, DH)).reshape(E2P, DW)

    pad0 = jnp.zeros((PAD,), jnp.int32)
    trash = N + (jnp.arange(PAD, dtype=jnp.int32) % (N_ACC - N))
    ga = jnp.concatenate([col, pad0, row + N, pad0]).reshape(E2P // 128, 128)
    gb = jnp.concatenate([row, pad0, col + N, pad0]).reshape(E2P // 128, 128)
    sc2 = jnp.concatenate([col, trash, row, trash]).reshape(E2P // 128, 128)

    s_part = _sc_agg(xi_s, xj_s, ae, ga, gb, sc2)
    acc, us, dis = _tc_postagg(s_part, W2, b2.reshape(1, DH),
                               tag0_W[0], isd)

    out = None
    tags = [(tag0_W, tag0_b), (tag1_W, tag1_b), (tag2_W, tag2_b)]
    for layer, (Ws, b) in enumerate(tags):
        for k in (1, 2, 3):
            sp = _sc_spmv(us, gb, sc2)
            if k < 3:
                acc, us = _tc_tag_mid(sp, dis, Ws[k], acc, isd)
            elif layer < 2:
                acc, us = _tc_tag_last(sp, dis, Ws[k], acc,
                                       b.reshape(1, DH),
                                       tags[layer + 1][0][0], isd)
            else:
                out = _tc_tag_final(sp, dis, Ws[k], acc, b.reshape(1, DH))
    return out


# speedup vs baseline: 11.4150x; 2.2671x over previous
import functools
import jax
import jax.numpy as jnp
from jax import lax
from jax.experimental import pallas as pl
from jax.experimental.pallas import tpu as pltpu
from jax.experimental.pallas import tpu_sc as plsc

N = 10000
E = 320000
DN = 128
DE = 16
DH = 64
DW = 128
DA = 80
NC = 2
NS = 16
L = 16
NW = NC * NS
PAD = 7680
EH = E + PAD
E2P = 2 * EH
PER_W = E2P // NW
IR_W = PER_W // 128
N_ACC = 10240
ROWS_PER_TILE = N_ACC // NS
CHUNK = 1024
NCHUNK = PER_W // CHUNK

_f32 = jnp.float32


def _tc_isdir(ei3):
    def body(ei_ref, o_ref):
        src0 = ei_ref[0, 0, 0]
        tgt0 = ei_ref[1, 0, 0]
        m = jnp.logical_and(ei_ref[0] == tgt0, ei_ref[1] == src0)
        red = jnp.max(jnp.where(m, 1.0, 0.0))
        o_ref[...] = (1.0 - red).reshape(1, 1)

    return pl.pallas_call(
        body, out_shape=jax.ShapeDtypeStruct((1, 1), _f32))(ei3)


def _tc_nodes(x, w1a, w1b, isd):
    def body(x_ref, wa_ref, wb_ref, isd_ref, xi_out, xj_out):
        s = isd_ref[0, 0]
        xi = jnp.dot(x_ref[...], wa_ref[...], preferred_element_type=_f32)
        xj = jnp.dot(x_ref[...], wb_ref[...], preferred_element_type=_f32)
        z = jnp.zeros((N, DW - DH), _f32)
        xi_out[pl.ds(0, N), pl.ds(0, DH)] = xi
        xi_out[pl.ds(N, N), pl.ds(0, DH)] = xi * s
        xi_out[pl.ds(0, N), pl.ds(DH, DW - DH)] = z
        xi_out[pl.ds(N, N), pl.ds(DH, DW - DH)] = z
        xj_out[pl.ds(0, N), pl.ds(0, DH)] = xj
        xj_out[pl.ds(N, N), pl.ds(0, DH)] = xj * s
        xj_out[pl.ds(0, N), pl.ds(DH, DW - DH)] = z
        xj_out[pl.ds(N, N), pl.ds(DH, DW - DH)] = z

    return pl.pallas_call(
        body,
        in_specs=[
            pl.BlockSpec(memory_space=pltpu.VMEM),
            pl.BlockSpec(memory_space=pltpu.VMEM),
            pl.BlockSpec(memory_space=pltpu.VMEM),
            pl.BlockSpec(memory_space=pltpu.SMEM),
        ],
        out_specs=[pl.BlockSpec(memory_space=pltpu.VMEM),
                   pl.BlockSpec(memory_space=pltpu.VMEM)],
        out_shape=[jax.ShapeDtypeStruct((2 * N, DW), _f32),
                   jax.ShapeDtypeStruct((2 * N, DW), _f32)],
    )(x, w1a, w1b, isd)


def _tc_ae(isd, ea_pad, wef, b1r):
    blk = 4096
    nblk = EH // blk

    def body(isd_ref, ea_ref, w_ref, b_ref, o_ref):
        h = pl.program_id(0)
        s = jnp.where(h == 0, 1.0, isd_ref[0, 0])
        lo = (jnp.dot(ea_ref[...], w_ref[0],
                      preferred_element_type=_f32) + b_ref[...]) * s
        wcol = jnp.where(
            lax.broadcasted_iota(jnp.int32, (blk, DA - DH), 1) == 0, s, 0.0)
        o_ref[0, :, pl.ds(0, DH)] = lo
        o_ref[0, :, pl.ds(DH, DA - DH)] = wcol

    return pl.pallas_call(
        body,
        grid=(2, nblk),
        in_specs=[
            pl.BlockSpec(memory_space=pltpu.SMEM),
            pl.BlockSpec((blk, DE), lambda h, j: (j, 0)),
            pl.BlockSpec((1, DE, DH), lambda h, j: (h, 0, 0)),
            pl.BlockSpec((1, DH), lambda h, j: (0, 0)),
        ],
        out_specs=pl.BlockSpec((1, blk, DW), lambda h, j: (h, j, 0)),
        out_shape=jax.ShapeDtypeStruct((2, EH, DW), _f32),
    )(isd, ea_pad, wef, b1r)


def _tc_postagg(s_part, w2, b2r, w0, isd):
    def body(s_ref, w2_ref, b2_ref, w0_ref, isd_ref,
             acc_out, us_out, dis_out):
        sv = s_ref[...]
        S = sv[0, :N, :DH] + sv[1, :N, :DH]
        deg = sv[0, :N, DH:DH + 1] + sv[1, :N, DH:DH + 1]
        h = jnp.dot(S, w2_ref[...], preferred_element_type=_f32) \
            + deg * b2_ref[...]
        dis = jnp.where(deg > 0, lax.rsqrt(jnp.where(deg > 0, deg, 1.0)), 0.0)
        acc_out[...] = jnp.dot(h, w0_ref[...], preferred_element_type=_f32)
        u = dis * h
        z = jnp.zeros((N, DW - DH), _f32)
        us_out[pl.ds(0, N), pl.ds(0, DH)] = u
        us_out[pl.ds(N, N), pl.ds(0, DH)] = u * isd_ref[0, 0]
        us_out[pl.ds(0, N), pl.ds(DH, DW - DH)] = z
        us_out[pl.ds(N, N), pl.ds(DH, DW - DH)] = z
        dis_out[...] = dis

    return pl.pallas_call(
        body,
        in_specs=[pl.BlockSpec(memory_space=pltpu.VMEM)] * 4
        + [pl.BlockSpec(memory_space=pltpu.SMEM)],
        out_specs=[pl.BlockSpec(memory_space=pltpu.VMEM)] * 3,
        out_shape=[jax.ShapeDtypeStruct((N, DH), _f32),
                   jax.ShapeDtypeStruct((2 * N, DW), _f32),
                   jax.ShapeDtypeStruct((N, 1), _f32)],
    )(s_part, w2, b2r, w0, isd)


def _tc_tag_mid(s_part, dis, wk, acc, isd):
    def body(s_ref, dis_ref, wk_ref, acc_ref, isd_ref, acc_out, us_out):
        sv = s_ref[...]
        dv = dis_ref[...]
        hk = dv * (sv[0, :N, :DH] + sv[1, :N, :DH])
        acc_out[...] = acc_ref[...] + jnp.dot(
            hk, wk_ref[...], preferred_element_type=_f32)
        u = dv * hk
        z = jnp.zeros((N, DW - DH), _f32)
        us_out[pl.ds(0, N), pl.ds(0, DH)] = u
        us_out[pl.ds(N, N), pl.ds(0, DH)] = u * isd_ref[0, 0]
        us_out[pl.ds(0, N), pl.ds(DH, DW - DH)] = z
        us_out[pl.ds(N, N), pl.ds(DH, DW - DH)] = z

    return pl.pallas_call(
        body,
        in_specs=[pl.BlockSpec(memory_space=pltpu.VMEM)] * 4
        + [pl.BlockSpec(memory_space=pltpu.SMEM)],
        out_specs=[pl.BlockSpec(memory_space=pltpu.VMEM)] * 2,
        out_shape=[jax.ShapeDtypeStruct((N, DH), _f32),
                   jax.ShapeDtypeStruct((2 * N, DW), _f32)],
    )(s_part, dis, wk, acc, isd)


def _tc_tag_last(s_part, dis, wk, acc, br, w0n, isd):
    def body(s_ref, dis_ref, wk_ref, acc_ref, b_ref, w0_ref, isd_ref,
             acc_out, us_out):
        sv = s_ref[...]
        dv = dis_ref[...]
        hk = dv * (sv[0, :N, :DH] + sv[1, :N, :DH])
        g = acc_ref[...] + jnp.dot(hk, wk_ref[...],
                                   preferred_element_type=_f32) + b_ref[...]
        hn = jnp.maximum(g, 0.0)
        acc_out[...] = jnp.dot(hn, w0_ref[...], preferred_element_type=_f32)
        u = dv * hn
        z = jnp.zeros((N, DW - DH), _f32)
        us_out[pl.ds(0, N), pl.ds(0, DH)] = u
        us_out[pl.ds(N, N), pl.ds(0, DH)] = u * isd_ref[0, 0]
        us_out[pl.ds(0, N), pl.ds(DH, DW - DH)] = z
        us_out[pl.ds(N, N), pl.ds(DH, DW - DH)] = z

    return pl.pallas_call(
        body,
        in_specs=[pl.BlockSpec(memory_space=pltpu.VMEM)] * 6
        + [pl.BlockSpec(memory_space=pltpu.SMEM)],
        out_specs=[pl.BlockSpec(memory_space=pltpu.VMEM)] * 2,
        out_shape=[jax.ShapeDtypeStruct((N, DH), _f32),
                   jax.ShapeDtypeStruct((2 * N, DW), _f32)],
    )(s_part, dis, wk, acc, br, w0n, isd)


def _tc_tag_final(s_part, dis, wk, acc, br):
    def body(s_ref, dis_ref, wk_ref, acc_ref, b_ref, o_ref):
        sv = s_ref[...]
        hk = dis_ref[...] * (sv[0, :N, :DH] + sv[1, :N, :DH])
        o_ref[...] = acc_ref[...] + jnp.dot(
            hk, wk_ref[...], preferred_element_type=_f32) + b_ref[...]

    return pl.pallas_call(
        body,
        in_specs=[pl.BlockSpec(memory_space=pltpu.VMEM)] * 5,
        out_specs=pl.BlockSpec(memory_space=pltpu.VMEM),
        out_shape=jax.ShapeDtypeStruct((N, DH), _f32),
    )(s_part, dis, wk, acc, br)


def _mesh():
    return plsc.VectorSubcoreMesh(
        core_axis_name="c", subcore_axis_name="s")


def _zero_acc(buf, acc, tz):
    def zrow(r, _):
        for sl in range(DW // L):
            buf[r, pl.ds(sl * L, L)] = jnp.zeros((L,), _f32)
        return 0
    lax.fori_loop(0, 128, zrow, 0, unroll=8)
    for j in range(ROWS_PER_TILE // 128):
        pltpu.sync_copy(buf, acc.at[pl.ds(tz + j * 128, 128)])


def _sc_agg(xi, xj, ae, ga, gb, sc2):
    @functools.partial(
        pl.kernel,
        out_type=jax.ShapeDtypeStruct((NC, N_ACC, DW), _f32),
        mesh=_mesh(),
        scratch_types=[
            pltpu.VMEM_SHARED((N_ACC, DW), _f32),
            pltpu.VMEM((8, 128), jnp.int32),
            pltpu.VMEM((8, 128), jnp.int32),
            pltpu.VMEM((8, 128), jnp.int32),
            pltpu.VMEM((128, DW), _f32),
            pltpu.VMEM((128, DW), _f32),
            pltpu.VMEM((64, DW), _f32),
            pltpu.SemaphoreType.DMA,
            pltpu.SemaphoreType.DMA,
        ],
    )
    def k(xi_h, xj_h, ae_h, ga_h, gb_h, sc2_h, s_out,
          s_acc, idxa, idxb, scx, rowsa, rowsb, aebuf, sem1, sem2):
        cid = lax.axis_index("c")
        sid = lax.axis_index("s")
        wid = cid * NS + sid
        base_row = wid * IR_W
        tz = sid * ROWS_PER_TILE

        _zero_acc(rowsa, s_acc, tz)
        plsc.subcore_barrier()

        def chunk(i, _):
            crow = base_row + i * (CHUNK // 128)
            pltpu.sync_copy(ga_h.at[pl.ds(crow, CHUNK // 128)], idxa)
            pltpu.sync_copy(gb_h.at[pl.ds(crow, CHUNK // 128)], idxb)
            pltpu.sync_copy(sc2_h.at[pl.ds(crow, CHUNK // 128)], scx)
            eoff = wid * PER_W + i * CHUNK
            for j in range(CHUNK // 128):
                d1 = pltpu.async_copy(xi_h.at[idxa.at[j]], rowsa, sem1)
                d2 = pltpu.async_copy(xj_h.at[idxb.at[j]], rowsb, sem2)
                d1.wait()
                d2.wait()

                for half in range(2):
                    pltpu.sync_copy(
                        ae_h.at[pl.ds(eoff + j * 128 + half * 64, 64)], aebuf)

                    def comp(r, _):
                        rr = half * 64 + r
                        for sl in range(DW // L):
                            s_ = pl.ds(sl * L, L)
                            rowsa[rr, s_] = jnp.maximum(
                                rowsa[rr, s_] + rowsb[rr, s_] + aebuf[r, s_],
                                0.0)
                        return 0
                    lax.fori_loop(0, 64, comp, 0, unroll=4)

                pltpu.sync_copy(rowsa, s_acc.at[scx.at[j]], add=True)
            return 0
        lax.fori_loop(0, NCHUNK, chunk, 0)

        plsc.subcore_barrier()
        pltpu.sync_copy(s_acc.at[pl.ds(tz, ROWS_PER_TILE)],
                        s_out.at[cid, pl.ds(tz, ROWS_PER_TILE)])

    return k(xi, xj, ae, ga, gb, sc2)


def _sc_spmv(us, gb, sc2):
    @functools.partial(
        pl.kernel,
        out_type=jax.ShapeDtypeStruct((NC, N_ACC, DW), _f32),
        mesh=_mesh(),
        scratch_types=[
            pltpu.VMEM_SHARED((N_ACC, DW), _f32),
            pltpu.VMEM((8, 128), jnp.int32),
            pltpu.VMEM((8, 128), jnp.int32),
            pltpu.VMEM((128, DW), _f32),
            pltpu.VMEM((128, DW), _f32),
            pltpu.SemaphoreType.DMA,
            pltpu.SemaphoreType.DMA,
        ],
    )
    def k(us_h, gb_h, sc2_h, s_out,
          s_acc, idxb, scx, rows0, rows1, sem0, sem1):
        cid = lax.axis_index("c")
        sid = lax.axis_index("s")
        wid = cid * NS + sid
        base_row = wid * IR_W
        tz = sid * ROWS_PER_TILE

        _zero_acc(rows0, s_acc, tz)
        plsc.subcore_barrier()

        def chunk(i, _):
            crow = base_row + i * (CHUNK // 128)
            pltpu.sync_copy(gb_h.at[pl.ds(crow, CHUNK // 128)], idxb)
            pltpu.sync_copy(sc2_h.at[pl.ds(crow, CHUNK // 128)], scx)
            bufs = (rows0, rows1)
            sems = (sem0, sem1)
            d0 = pltpu.async_copy(us_h.at[idxb.at[0]], rows0, sem0)
            dmas = [d0, None]
            for j in range(CHUNK // 128):
                b = j % 2
                nb = (j + 1) % 2
                if j + 1 < CHUNK // 128:
                    dmas[nb] = pltpu.async_copy(
                        us_h.at[idxb.at[j + 1]], bufs[nb], sems[nb])
                dmas[b].wait()
                pltpu.sync_copy(bufs[b], s_acc.at[scx.at[j]], add=True)
            return 0
        lax.fori_loop(0, NCHUNK, chunk, 0)

        plsc.subcore_barrier()
        pltpu.sync_copy(s_acc.at[pl.ds(tz, ROWS_PER_TILE)],
                        s_out.at[cid, pl.ds(tz, ROWS_PER_TILE)])

    return k(us, gb, sc2)


def kernel(x, edge_index, edge_attr, W1, b1, W2, b2,
           tag0_W, tag0_b, tag1_W, tag1_b, tag2_W, tag2_b):
    row, col = edge_index[0], edge_index[1]
    ei3 = edge_index.reshape(2, E // 128, 128)
    isd = _tc_isdir(ei3)

    w1a, w1b, w1e = W1[:DN], W1[DN:2 * DN], W1[2 * DN:]
    sgn = jnp.array([-1.0, 1.0, -1.0] + [1.0] * (DE - 3), _f32)[:, None]
    wef = jnp.stack([w1e, w1e * sgn])
    xi_s, xj_s = _tc_nodes(x, w1a, w1b, isd)
    ea_pad = jnp.zeros((EH, DE), _f32).at[:E].set(edge_attr)
    ae = _tc_ae(isd, ea_pad, wef, b1.reshape(1, DH)).reshape(E2P, DW)

    pad0 = jnp.arange(PAD, dtype=jnp.int32) % N
    trash = N + (jnp.arange(PAD, dtype=jnp.int32) % (N_ACC - N))
    ga = jnp.concatenate([col, pad0, row + N, pad0]).reshape(E2P // 128, 128)
    gb = jnp.concatenate([row, pad0, col + N, pad0]).reshape(E2P // 128, 128)
    sc2 = jnp.concatenate([col, trash, row, trash]).reshape(E2P // 128, 128)

    s_part = _sc_agg(xi_s, xj_s, ae, ga, gb, sc2)
    acc, us, dis = _tc_postagg(s_part, W2, b2.reshape(1, DH),
                               tag0_W[0], isd)

    out = None
    tags = [(tag0_W, tag0_b), (tag1_W, tag1_b), (tag2_W, tag2_b)]
    for layer, (Ws, b) in enumerate(tags):
        for k in (1, 2, 3):
            sp = _sc_spmv(us, gb, sc2)
            if k < 3:
                acc, us = _tc_tag_mid(sp, dis, Ws[k], acc, isd)
            elif layer < 2:
                acc, us = _tc_tag_last(sp, dis, Ws[k], acc,
                                       b.reshape(1, DH),
                                       tags[layer + 1][0][0], isd)
            else:
                out = _tc_tag_final(sp, dis, Ws[k], acc, b.reshape(1, DH))
    return out


# speedup vs baseline: 13.1757x; 1.1542x over previous
import functools
import jax
import jax.numpy as jnp
from jax import lax
from jax.experimental import pallas as pl
from jax.experimental.pallas import tpu as pltpu
from jax.experimental.pallas import tpu_sc as plsc

N = 10000
E = 320000
DN = 128
DE = 16
DH = 64
DW = 128
DA = 80
NC = 2
NS = 16
L = 16
NW = NC * NS
PAD = 7680
EH = E + PAD
E2P = 2 * EH
PER_W = E2P // NW
IR_W = PER_W // 128
N_ACC = 10240
ROWS_PER_TILE = N_ACC // NS
CHUNK = 1024
NCHUNK = PER_W // CHUNK

_f32 = jnp.float32


def _tc_isdir(ei3):
    def body(ei_ref, o_ref):
        src0 = ei_ref[0, 0, 0]
        tgt0 = ei_ref[1, 0, 0]
        m = jnp.logical_and(ei_ref[0] == tgt0, ei_ref[1] == src0)
        red = jnp.max(jnp.where(m, 1.0, 0.0))
        o_ref[...] = (1.0 - red).reshape(1, 1)

    return pl.pallas_call(
        body, out_shape=jax.ShapeDtypeStruct((1, 1), _f32))(ei3)


def _tc_nodes(x, w1a, w1b, isd):
    def body(x_ref, wa_ref, wb_ref, isd_ref, xi_out, xj_out):
        s = isd_ref[0, 0]
        xi = jnp.dot(x_ref[...], wa_ref[...], preferred_element_type=_f32)
        xj = jnp.dot(x_ref[...], wb_ref[...], preferred_element_type=_f32)
        z = jnp.zeros((N, DW - DH), _f32)
        xi_out[pl.ds(0, N), pl.ds(0, DH)] = xi
        xi_out[pl.ds(N, N), pl.ds(0, DH)] = xi * s
        xi_out[pl.ds(0, N), pl.ds(DH, DW - DH)] = z
        xi_out[pl.ds(N, N), pl.ds(DH, DW - DH)] = z
        xj_out[pl.ds(0, N), pl.ds(0, DH)] = xj
        xj_out[pl.ds(N, N), pl.ds(0, DH)] = xj * s
        xj_out[pl.ds(0, N), pl.ds(DH, DW - DH)] = z
        xj_out[pl.ds(N, N), pl.ds(DH, DW - DH)] = z

    return pl.pallas_call(
        body,
        in_specs=[
            pl.BlockSpec(memory_space=pltpu.VMEM),
            pl.BlockSpec(memory_space=pltpu.VMEM),
            pl.BlockSpec(memory_space=pltpu.VMEM),
            pl.BlockSpec(memory_space=pltpu.SMEM),
        ],
        out_specs=[pl.BlockSpec(memory_space=pltpu.VMEM),
                   pl.BlockSpec(memory_space=pltpu.VMEM)],
        out_shape=[jax.ShapeDtypeStruct((2 * N, DW), _f32),
                   jax.ShapeDtypeStruct((2 * N, DW), _f32)],
    )(x, w1a, w1b, isd)


def _tc_ae(isd, ea_pad, wef, b1r):
    blk = 4096
    nblk = EH // blk

    def body(isd_ref, ea_ref, w_ref, b_ref, o_ref):
        h = pl.program_id(0)
        s = jnp.where(h == 0, 1.0, isd_ref[0, 0])
        lo = (jnp.dot(ea_ref[...], w_ref[0],
                      preferred_element_type=_f32) + b_ref[...]) * s
        wcol = jnp.where(
            lax.broadcasted_iota(jnp.int32, (blk, DA - DH), 1) == 0, s, 0.0)
        o_ref[0, :, pl.ds(0, DH)] = lo
        o_ref[0, :, pl.ds(DH, DA - DH)] = wcol

    return pl.pallas_call(
        body,
        grid=(2, nblk),
        in_specs=[
            pl.BlockSpec(memory_space=pltpu.SMEM),
            pl.BlockSpec((blk, DE), lambda h, j: (j, 0)),
            pl.BlockSpec((1, DE, DH), lambda h, j: (h, 0, 0)),
            pl.BlockSpec((1, DH), lambda h, j: (0, 0)),
        ],
        out_specs=pl.BlockSpec((1, blk, DW), lambda h, j: (h, j, 0)),
        out_shape=jax.ShapeDtypeStruct((2, EH, DW), _f32),
    )(isd, ea_pad, wef, b1r)


def _tc_postagg(s_part, w2, b2r, w0, isd):
    def body(s_ref, w2_ref, b2_ref, w0_ref, isd_ref,
             acc_out, us_out, dis_out):
        sv = s_ref[...]
        S = sv[0, :N, :DH] + sv[1, :N, :DH]
        deg = sv[0, :N, DH:DH + 1] + sv[1, :N, DH:DH + 1]
        h = jnp.dot(S, w2_ref[...], preferred_element_type=_f32) \
            + deg * b2_ref[...]
        dis = jnp.where(deg > 0, lax.rsqrt(jnp.where(deg > 0, deg, 1.0)), 0.0)
        acc_out[...] = jnp.dot(h, w0_ref[...], preferred_element_type=_f32)
        u = dis * h
        z = jnp.zeros((N, DW - DH), _f32)
        us_out[pl.ds(0, N), pl.ds(0, DH)] = u
        us_out[pl.ds(N, N), pl.ds(0, DH)] = u * isd_ref[0, 0]
        us_out[pl.ds(0, N), pl.ds(DH, DW - DH)] = z
        us_out[pl.ds(N, N), pl.ds(DH, DW - DH)] = z
        dis_out[...] = dis

    return pl.pallas_call(
        body,
        in_specs=[pl.BlockSpec(memory_space=pltpu.VMEM)] * 4
        + [pl.BlockSpec(memory_space=pltpu.SMEM)],
        out_specs=[pl.BlockSpec(memory_space=pltpu.VMEM)] * 3,
        out_shape=[jax.ShapeDtypeStruct((N, DH), _f32),
                   jax.ShapeDtypeStruct((2 * N, DW), _f32),
                   jax.ShapeDtypeStruct((N, 1), _f32)],
    )(s_part, w2, b2r, w0, isd)


def _tc_tag_mid(s_part, dis, wk, acc, isd):
    def body(s_ref, dis_ref, wk_ref, acc_ref, isd_ref, acc_out, us_out):
        sv = s_ref[...]
        dv = dis_ref[...]
        hk = dv * (sv[0, :N, :DH] + sv[1, :N, :DH])
        acc_out[...] = acc_ref[...] + jnp.dot(
            hk, wk_ref[...], preferred_element_type=_f32)
        u = dv * hk
        z = jnp.zeros((N, DW - DH), _f32)
        us_out[pl.ds(0, N), pl.ds(0, DH)] = u
        us_out[pl.ds(N, N), pl.ds(0, DH)] = u * isd_ref[0, 0]
        us_out[pl.ds(0, N), pl.ds(DH, DW - DH)] = z
        us_out[pl.ds(N, N), pl.ds(DH, DW - DH)] = z

    return pl.pallas_call(
        body,
        in_specs=[pl.BlockSpec(memory_space=pltpu.VMEM)] * 4
        + [pl.BlockSpec(memory_space=pltpu.SMEM)],
        out_specs=[pl.BlockSpec(memory_space=pltpu.VMEM)] * 2,
        out_shape=[jax.ShapeDtypeStruct((N, DH), _f32),
                   jax.ShapeDtypeStruct((2 * N, DW), _f32)],
    )(s_part, dis, wk, acc, isd)


def _tc_tag_last(s_part, dis, wk, acc, br, w0n, isd):
    def body(s_ref, dis_ref, wk_ref, acc_ref, b_ref, w0_ref, isd_ref,
             acc_out, us_out):
        sv = s_ref[...]
        dv = dis_ref[...]
        hk = dv * (sv[0, :N, :DH] + sv[1, :N, :DH])
        g = acc_ref[...] + jnp.dot(hk, wk_ref[...],
                                   preferred_element_type=_f32) + b_ref[...]
        hn = jnp.maximum(g, 0.0)
        acc_out[...] = jnp.dot(hn, w0_ref[...], preferred_element_type=_f32)
        u = dv * hn
        z = jnp.zeros((N, DW - DH), _f32)
        us_out[pl.ds(0, N), pl.ds(0, DH)] = u
        us_out[pl.ds(N, N), pl.ds(0, DH)] = u * isd_ref[0, 0]
        us_out[pl.ds(0, N), pl.ds(DH, DW - DH)] = z
        us_out[pl.ds(N, N), pl.ds(DH, DW - DH)] = z

    return pl.pallas_call(
        body,
        in_specs=[pl.BlockSpec(memory_space=pltpu.VMEM)] * 6
        + [pl.BlockSpec(memory_space=pltpu.SMEM)],
        out_specs=[pl.BlockSpec(memory_space=pltpu.VMEM)] * 2,
        out_shape=[jax.ShapeDtypeStruct((N, DH), _f32),
                   jax.ShapeDtypeStruct((2 * N, DW), _f32)],
    )(s_part, dis, wk, acc, br, w0n, isd)


def _tc_tag_final(s_part, dis, wk, acc, br):
    def body(s_ref, dis_ref, wk_ref, acc_ref, b_ref, o_ref):
        sv = s_ref[...]
        hk = dis_ref[...] * (sv[0, :N, :DH] + sv[1, :N, :DH])
        o_ref[...] = acc_ref[...] + jnp.dot(
            hk, wk_ref[...], preferred_element_type=_f32) + b_ref[...]

    return pl.pallas_call(
        body,
        in_specs=[pl.BlockSpec(memory_space=pltpu.VMEM)] * 5,
        out_specs=pl.BlockSpec(memory_space=pltpu.VMEM),
        out_shape=jax.ShapeDtypeStruct((N, DH), _f32),
    )(s_part, dis, wk, acc, br)


def _mesh():
    return plsc.VectorSubcoreMesh(
        core_axis_name="c", subcore_axis_name="s")


def _zero_acc(buf, acc, tz):
    def zrow(r, _):
        for sl in range(DW // L):
            buf[r, pl.ds(sl * L, L)] = jnp.zeros((L,), _f32)
        return 0
    lax.fori_loop(0, 128, zrow, 0, unroll=8)
    for j in range(ROWS_PER_TILE // 128):
        pltpu.sync_copy(buf, acc.at[pl.ds(tz + j * 128, 128)])


def _sc_agg(xi, xj, ae, ga, gb, sc2, aeidx):
    @functools.partial(
        pl.kernel,
        out_type=jax.ShapeDtypeStruct((NC, N_ACC, DW), _f32),
        mesh=_mesh(),
        scratch_types=[
            pltpu.VMEM_SHARED((N_ACC, DW), _f32),
            pltpu.VMEM((8, 128), jnp.int32),
            pltpu.VMEM((8, 128), jnp.int32),
            pltpu.VMEM((8, 128), jnp.int32),
            pltpu.VMEM((8, 128), jnp.int32),
            pltpu.VMEM((128, DW), _f32),
            pltpu.VMEM((128, DW), _f32),
            pltpu.SemaphoreType.DMA,
            pltpu.SemaphoreType.DMA,
            pltpu.SemaphoreType.DMA,
        ],
    )
    def k(xi_h, xj_h, ae_h, ga_h, gb_h, sc2_h, aeidx_h, s_out,
          s_acc, idxa, idxb, scx, idxe, rowsa, rowsb, sem1, sem2, sem3):
        cid = lax.axis_index("c")
        sid = lax.axis_index("s")
        wid = cid * NS + sid
        base_row = wid * IR_W
        tz = sid * ROWS_PER_TILE

        _zero_acc(rowsa, s_acc, tz)
        plsc.subcore_barrier()

        def chunk(i, _):
            crow = base_row + i * (CHUNK // 128)
            pltpu.sync_copy(ga_h.at[pl.ds(crow, CHUNK // 128)], idxa)
            pltpu.sync_copy(gb_h.at[pl.ds(crow, CHUNK // 128)], idxb)
            pltpu.sync_copy(sc2_h.at[pl.ds(crow, CHUNK // 128)], scx)
            pltpu.sync_copy(aeidx_h.at[pl.ds(crow, CHUNK // 128)], idxe)
            for j in range(CHUNK // 128):
                d1 = pltpu.async_copy(xi_h.at[idxa.at[j]], rowsa, sem1)
                d2 = pltpu.async_copy(xj_h.at[idxb.at[j]], rowsb, sem2)
                d2.wait()
                d3 = pltpu.async_copy(ae_h.at[idxe.at[j]], rowsb, sem3,
                                      add=True)
                d3.wait()
                d1.wait()

                def comp(r, _):
                    for sl in range(DA // L):
                        s_ = pl.ds(sl * L, L)
                        rowsa[r, s_] = jnp.maximum(
                            rowsa[r, s_] + rowsb[r, s_], 0.0)
                    return 0
                lax.fori_loop(0, 128, comp, 0, unroll=4)

                pltpu.sync_copy(rowsa, s_acc.at[scx.at[j]], add=True)
            return 0
        lax.fori_loop(0, NCHUNK, chunk, 0)

        plsc.subcore_barrier()
        pltpu.sync_copy(s_acc.at[pl.ds(tz, ROWS_PER_TILE)],
                        s_out.at[cid, pl.ds(tz, ROWS_PER_TILE)])

    return k(xi, xj, ae, ga, gb, sc2, aeidx)


def _sc_spmv(us, gb, sc2):
    @functools.partial(
        pl.kernel,
        out_type=jax.ShapeDtypeStruct((NC, N_ACC, DW), _f32),
        mesh=_mesh(),
        scratch_types=[
            pltpu.VMEM_SHARED((N_ACC, DW), _f32),
            pltpu.VMEM((8, 128), jnp.int32),
            pltpu.VMEM((8, 128), jnp.int32),
            pltpu.VMEM((128, DW), _f32),
            pltpu.VMEM((128, DW), _f32),
            pltpu.SemaphoreType.DMA,
            pltpu.SemaphoreType.DMA,
        ],
    )
    def k(us_h, gb_h, sc2_h, s_out,
          s_acc, idxb, scx, rows0, rows1, sem0, sem1):
        cid = lax.axis_index("c")
        sid = lax.axis_index("s")
        wid = cid * NS + sid
        base_row = wid * IR_W
        tz = sid * ROWS_PER_TILE

        _zero_acc(rows0, s_acc, tz)
        plsc.subcore_barrier()

        def chunk(i, _):
            crow = base_row + i * (CHUNK // 128)
            pltpu.sync_copy(gb_h.at[pl.ds(crow, CHUNK // 128)], idxb)
            pltpu.sync_copy(sc2_h.at[pl.ds(crow, CHUNK // 128)], scx)
            bufs = (rows0, rows1)
            sems = (sem0, sem1)
            d0 = pltpu.async_copy(us_h.at[idxb.at[0]], rows0, sem0)
            dmas = [d0, None]
            for j in range(CHUNK // 128):
                b = j % 2
                nb = (j + 1) % 2
                if j + 1 < CHUNK // 128:
                    dmas[nb] = pltpu.async_copy(
                        us_h.at[idxb.at[j + 1]], bufs[nb], sems[nb])
                dmas[b].wait()
                pltpu.sync_copy(bufs[b], s_acc.at[scx.at[j]], add=True)
            return 0
        lax.fori_loop(0, NCHUNK, chunk, 0)

        plsc.subcore_barrier()
        pltpu.sync_copy(s_acc.at[pl.ds(tz, ROWS_PER_TILE)],
                        s_out.at[cid, pl.ds(tz, ROWS_PER_TILE)])

    return k(us, gb, sc2)


def kernel(x, edge_index, edge_attr, W1, b1, W2, b2,
           tag0_W, tag0_b, tag1_W, tag1_b, tag2_W, tag2_b):
    row, col = edge_index[0], edge_index[1]
    ei3 = edge_index.reshape(2, E // 128, 128)
    isd = _tc_isdir(ei3)

    w1a, w1b, w1e = W1[:DN], W1[DN:2 * DN], W1[2 * DN:]
    sgn = jnp.array([-1.0, 1.0, -1.0] + [1.0] * (DE - 3), _f32)[:, None]
    wef = jnp.stack([w1e, w1e * sgn])
    xi_s, xj_s = _tc_nodes(x, w1a, w1b, isd)
    ea_pad = jnp.zeros((EH, DE), _f32).at[:E].set(edge_attr)
    ae = _tc_ae(isd, ea_pad, wef, b1.reshape(1, DH)).reshape(E2P, DW)

    pad0 = jnp.arange(PAD, dtype=jnp.int32) % N
    trash = N + (jnp.arange(PAD, dtype=jnp.int32) % (N_ACC - N))
    ga = jnp.concatenate([col, pad0, row + N, pad0]).reshape(E2P // 128, 128)
    gb = jnp.concatenate([row, pad0, col + N, pad0]).reshape(E2P // 128, 128)
    sc2 = jnp.concatenate([col, trash, row, trash]).reshape(E2P // 128, 128)
    aeidx = jnp.arange(E2P, dtype=jnp.int32).reshape(E2P // 128, 128)

    s_part = _sc_agg(xi_s, xj_s, ae, ga, gb, sc2, aeidx)
    acc, us, dis = _tc_postagg(s_part, W2, b2.reshape(1, DH),
                               tag0_W[0], isd)

    out = None
    tags = [(tag0_W, tag0_b), (tag1_W, tag1_b), (tag2_W, tag2_b)]
    for layer, (Ws, b) in enumerate(tags):
        for k in (1, 2, 3):
            sp = _sc_spmv(us, gb, sc2)
            if k < 3:
                acc, us = _tc_tag_mid(sp, dis, Ws[k], acc, isd)
            elif layer < 2:
                acc, us = _tc_tag_last(sp, dis, Ws[k], acc,
                                       b.reshape(1, DH),
                                       tags[layer + 1][0][0], isd)
            else:
                out = _tc_tag_final(sp, dis, Ws[k], acc, b.reshape(1, DH))
    return out
